# Initial kernel scaffold; baseline (speedup 1.0000x reference)
import jax
import jax.numpy as jnp
from jax.experimental import pallas as pl


def kernel(seq_len, pe_k, W, b):
    raise NotImplementedError("write your pallas kernel here")


# speedup vs baseline: 8.8135x; 8.8135x over previous
import jax
import jax.numpy as jnp
from jax.experimental import pallas as pl
from jax.experimental.pallas import tpu as pltpu

S = 2048
E = 32
ROWS_PER_BLOCK = 16


def _rpe_body(pe_ref, w_ref, b_ref, out_ref, proj_ref):
    i = pl.program_id(0)

    @pl.when(i == 0)
    def _():
        proj_ref[...] = (
            jnp.dot(pe_ref[...], w_ref[...].T, preferred_element_type=jnp.float32)
            + b_ref[...]
        )

    base = S - i * ROWS_PER_BLOCK
    for r in range(ROWS_PER_BLOCK):
        out_ref[r] = proj_ref[pl.ds(base - r, S), :]


def kernel(seq_len, pe_k, W, b):
    del seq_len
    grid = S // ROWS_PER_BLOCK
    return pl.pallas_call(
        _rpe_body,
        grid=(grid,),
        in_specs=[
            pl.BlockSpec((2 * S, E), lambda i: (0, 0)),
            pl.BlockSpec((E, E), lambda i: (0, 0)),
            pl.BlockSpec((1, E), lambda i: (0, 0)),
        ],
        out_specs=pl.BlockSpec((ROWS_PER_BLOCK, S, E), lambda i: (i, 0, 0)),
        out_shape=jax.ShapeDtypeStruct((S, S, E), jnp.float32),
        scratch_shapes=[pltpu.VMEM((2 * S, E), jnp.float32)],
    )(pe_k, W, jnp.reshape(b, (1, E)))


# speedup vs baseline: 16.1133x; 1.8283x over previous
import jax
import jax.numpy as jnp
from jax.experimental import pallas as pl
from jax.experimental.pallas import tpu as pltpu

S = 2048
E = 32
FLAT = 2 * S * E
ROW_BLOCK = 64


def _proj_body(pe_ref, w_ref, b_ref, o_ref):
    o_ref[...] = (
        jnp.dot(pe_ref[...], w_ref[...].T, preferred_element_type=jnp.float32)
        + b_ref[...]
    )


def _expand_body(flat_ref, out_ref, b2_ref):
    i = pl.program_id(0)

    @pl.when(i == 0)
    def _():
        for s in range(8):
            b2_ref[s, pl.ds(E * s, FLAT - E * s)] = flat_ref[0, pl.ds(0, FLAT - E * s)]

    base = S * E - E * ROW_BLOCK * i
    for o in range(ROW_BLOCK // 8):
        out_ref[pl.ds(8 * o, 8), :] = b2_ref[:, pl.ds(base - 8 * E * o, S * E)]


def kernel(seq_len, pe_k, W, b):
    del seq_len

    proj = pl.pallas_call(
        _proj_body,
        out_shape=jax.ShapeDtypeStruct((2 * S, E), jnp.float32),
    )(pe_k, W, jnp.reshape(b, (1, E)))

    flat = jnp.reshape(proj, (1, FLAT))

    out2d = pl.pallas_call(
        _expand_body,
        grid=(S // ROW_BLOCK,),
        in_specs=[pl.BlockSpec((1, FLAT), lambda i: (0, 0))],
        out_specs=pl.BlockSpec((ROW_BLOCK, S * E), lambda i: (i, 0)),
        out_shape=jax.ShapeDtypeStruct((S, S * E), jnp.float32),
        scratch_shapes=[pltpu.VMEM((8, FLAT), jnp.float32)],
    )(flat)

    return jnp.reshape(out2d, (S, S, E))
